# baseline (speedup 1.0000x reference)
import functools
import math

import jax
import jax.numpy as jnp
from jax import lax
from jax.experimental import pallas as pl
from jax.experimental.pallas import tpu as pltpu
from jax.experimental.pallas import tpu_sc as plsc

D_MODEL = 64
SCALE = 8.0
LANES = 16
CH = 128
NB = 3


def _make_sc_gather(B, V):
    info = plsc.get_sparse_core_info()
    NC, NS = info.num_cores, info.num_subcores
    NW = NC * NS
    per_w = B // NW
    nch = per_w // CH

    mesh = plsc.VectorSubcoreMesh(core_axis_name="c", subcore_axis_name="s")

    @functools.partial(
        pl.kernel,
        out_type=jax.ShapeDtypeStruct((B, D_MODEL), jnp.float32),
        mesh=mesh,
        scratch_types=[
            pltpu.VMEM((nch, CH), jnp.int32),
            [pltpu.VMEM((CH,), jnp.int32) for _ in range(NB)],
            [pltpu.VMEM((CH,), jnp.int32) for _ in range(NB)],
            [pltpu.VMEM((CH, 2 * D_MODEL), jnp.float32) for _ in range(NB)],
            [pltpu.VMEM((CH, D_MODEL), jnp.float32) for _ in range(NB)],
            [pltpu.SemaphoreType.DMA for _ in range(NB)],
            [pltpu.SemaphoreType.DMA for _ in range(NB)],
        ],
        compiler_params=pltpu.CompilerParams(needs_layout_passes=False),
    )
    def body(tab_hbm, idx_hbm, out_hbm, idx_all, p_slots, h_slots, bufs,
             obufs, gsems, ssems):
        wid = lax.axis_index("s") * NC + lax.axis_index("c")
        wrow = wid * per_w

        pltpu.sync_copy(idx_hbm.at[pl.ds(wid * nch, nch)], idx_all)

        def gather_desc(s):
            return pltpu.make_async_copy(
                tab_hbm.at[p_slots[s]], bufs[s], gsems[s]
            )

        def scatter_desc(c, s):
            return pltpu.make_async_copy(
                obufs[s], out_hbm.at[pl.ds(wrow + c * CH, CH)], ssems[s]
            )

        def pre(c, s):
            @pl.when(c >= NB)
            def _():
                scatter_desc(c - NB, s).wait()

            for g in range(CH // LANES):
                sl = pl.ds(g * LANES, LANES)
                v = idx_all[c, sl]
                p_slots[s][sl] = v >> 1
                h_slots[s][sl] = (v & 1) << 6
            gather_desc(s).start()

        def post(c, s):
            gather_desc(s).wait()
            buf, obuf = bufs[s], obufs[s]
            for g in range(CH // LANES):
                rowv = lax.iota(jnp.int32, LANES) + (g * LANES)
                hv = h_slots[s][pl.ds(g * LANES, LANES)]

                @plsc.parallel_loop(0, D_MODEL, unroll=4)
                def _sel(cc):
                    ccv = jnp.full((LANES,), cc, jnp.int32)
                    src = plsc.load_gather(buf, [rowv, hv + ccv])
                    plsc.store_scatter(obuf, [rowv, ccv], src * SCALE)

            scatter_desc(c, s).start()

        def step(c, s_pre, s_post):
            @pl.when(c < nch)
            def _():
                pre(c, s_pre)

            c2 = c - 2

            @pl.when(jnp.logical_and(c2 >= 0, c2 < nch))
            def _():
                post(c2, s_post)

        n_steps = nch + 2
        n_rounds = (n_steps + NB - 1) // NB

        def round_body(k, carry):
            for t in range(NB):
                step(k * NB + t, t, (t + 1) % NB)
            return carry

        lax.fori_loop(0, n_rounds, round_body, 0)

        for c in range(nch - NB, nch):
            scatter_desc(c, c % NB).wait()

    return body


def kernel(token_ids, embedding_weights):
    BATCH, HIST = token_ids.shape
    B = BATCH * HIST
    V = embedding_weights.shape[0]
    tab2 = embedding_weights.reshape(V // 2, 2 * D_MODEL)
    idx = token_ids.reshape(B // CH, CH)
    out = _make_sc_gather(B, V)(tab2, idx)
    return out.reshape(BATCH, HIST, D_MODEL)


# speedup vs baseline: 1.3729x; 1.3729x over previous
import functools
import math

import jax
import jax.numpy as jnp
from jax import lax
from jax.experimental import pallas as pl
from jax.experimental.pallas import tpu as pltpu
from jax.experimental.pallas import tpu_sc as plsc

D_MODEL = 64
SCALE = 8.0
LANES = 16
CT = 128
NB = 4


def _make_sc_gather(BATCH, HIST, V):
    info = plsc.get_sparse_core_info()
    NC, NS = info.num_cores, info.num_subcores
    NW = NC * NS
    assert BATCH % (CT * NW) == 0
    nch = HIST
    tg = CT // LANES

    mesh = plsc.VectorSubcoreMesh(core_axis_name="c", subcore_axis_name="s")

    @functools.partial(
        pl.kernel,
        out_type=jax.ShapeDtypeStruct((HIST, D_MODEL, BATCH), jnp.float32),
        mesh=mesh,
        scratch_types=[
            pltpu.VMEM((nch, CT), jnp.int32),
            [pltpu.VMEM((CT,), jnp.int32) for _ in range(NB)],
            [pltpu.VMEM((CT,), jnp.int32) for _ in range(NB)],
            [pltpu.VMEM((CT, 2 * D_MODEL), jnp.float32) for _ in range(NB)],
            [pltpu.VMEM((D_MODEL, CT), jnp.float32) for _ in range(NB)],
            [pltpu.SemaphoreType.DMA for _ in range(NB)],
            [pltpu.SemaphoreType.DMA for _ in range(NB)],
        ],
        compiler_params=pltpu.CompilerParams(needs_layout_passes=False),
    )
    def body(tab_hbm, idx_hbm, out_hbm, idx_all, pvs, hvs, bufs, tbufs,
             gsems, ssems):
        wid = lax.axis_index("s") * NC + lax.axis_index("c")
        bcol = wid * CT

        for th in range(nch // 8):
            pltpu.sync_copy(
                idx_hbm.at[pl.ds(th * 8, 8), pl.ds(bcol, CT)],
                idx_all.at[pl.ds(th * 8, 8)],
            )

        def gather_desc(s):
            return pltpu.make_async_copy(tab_hbm.at[pvs[s]], bufs[s], gsems[s])

        def scatter_descs(h, s):
            return [
                pltpu.make_async_copy(
                    tbufs[s].at[pl.ds(tc * 8, 8)],
                    out_hbm.at[h, pl.ds(tc * 8, 8), pl.ds(bcol, CT)],
                    ssems[s],
                )
                for tc in range(D_MODEL // 8)
            ]

        def pre(h, s):
            @pl.when(h >= NB)
            def _():
                for d in scatter_descs(h - NB, s):
                    d.wait()

            for g in range(tg):
                sl = pl.ds(g * LANES, LANES)
                v = idx_all[h, sl]
                pvs[s][sl] = v >> 1
                hvs[s][sl] = (v & 1) << 6
            gather_desc(s).start()

        def post(h, s):
            gather_desc(s).wait()
            buf, tbuf, hv_ref = bufs[s], tbufs[s], hvs[s]
            for g in range(tg):
                rowv = lax.iota(jnp.int32, LANES) + (g * LANES)
                hv = hv_ref[pl.ds(g * LANES, LANES)]

                @plsc.parallel_loop(0, D_MODEL, unroll=4)
                def _sel(c):
                    cv = jnp.full((LANES,), c, jnp.int32)
                    v = plsc.load_gather(buf, [rowv, hv + cv])
                    tbuf[c, pl.ds(g * LANES, LANES)] = v * SCALE

            for d in scatter_descs(h, s):
                d.start()

        def step(h, s_pre, s_post):
            @pl.when(h < nch)
            def _():
                pre(h, s_pre)

            h2 = h - 2

            @pl.when(jnp.logical_and(h2 >= 0, h2 < nch))
            def _():
                post(h2, s_post)

        n_steps = nch + 2
        n_rounds = (n_steps + NB - 1) // NB

        def round_body(k, carry):
            for t in range(NB):
                step(k * NB + t, t, (t + 2) % NB)
            return carry

        lax.fori_loop(0, n_rounds, round_body, 0)

        for h in range(nch - NB, nch):
            for d in scatter_descs(h, h % NB):
                d.wait()

    return body


def kernel(token_ids, embedding_weights):
    BATCH, HIST = token_ids.shape
    V = embedding_weights.shape[0]
    tab2 = embedding_weights.reshape(V // 2, 2 * D_MODEL)
    idxT = token_ids.T
    out3 = _make_sc_gather(BATCH, HIST, V)(tab2, idxT)
    return out3.transpose(2, 0, 1)


# speedup vs baseline: 1.4007x; 1.0203x over previous
import functools
import math

import jax
import jax.numpy as jnp
from jax import lax
from jax.experimental import pallas as pl
from jax.experimental.pallas import tpu as pltpu
from jax.experimental.pallas import tpu_sc as plsc

D_MODEL = 64
SCALE = 8.0
LANES = 16
CT = 128
NB = 4


def _make_sc_gather(BATCH, HIST, V):
    info = plsc.get_sparse_core_info()
    NC, NS = info.num_cores, info.num_subcores
    NW = NC * NS
    assert BATCH % (CT * NW) == 0
    nch = HIST
    tg = CT // LANES

    mesh = plsc.VectorSubcoreMesh(core_axis_name="c", subcore_axis_name="s")

    @functools.partial(
        pl.kernel,
        out_type=jax.ShapeDtypeStruct((HIST, D_MODEL, BATCH), jnp.float32),
        mesh=mesh,
        scratch_types=[
            pltpu.VMEM((nch, CT), jnp.int32),
            [pltpu.VMEM((CT,), jnp.int32) for _ in range(NB)],
            [pltpu.VMEM((CT,), jnp.int32) for _ in range(NB)],
            [pltpu.VMEM((CT, 2 * D_MODEL), jnp.float32) for _ in range(NB)],
            [pltpu.VMEM((D_MODEL, CT), jnp.float32) for _ in range(NB)],
            [pltpu.SemaphoreType.DMA for _ in range(NB)],
            [pltpu.SemaphoreType.DMA for _ in range(NB)],
        ],
        compiler_params=pltpu.CompilerParams(needs_layout_passes=False),
    )
    def body(tab_hbm, idx_hbm, out_hbm, idx_all, pvs, hvs, bufs, tbufs,
             gsems, ssems):
        wid = lax.axis_index("s") * NC + lax.axis_index("c")
        bcol = wid * CT

        for th in range(nch // 8):
            pltpu.sync_copy(
                idx_hbm.at[pl.ds(th * 8, 8), pl.ds(bcol, CT)],
                idx_all.at[pl.ds(th * 8, 8)],
            )

        def gather_desc(s):
            return pltpu.make_async_copy(tab_hbm.at[pvs[s]], bufs[s], gsems[s])

        def scatter_descs(h, s):
            return [
                pltpu.make_async_copy(
                    tbufs[s].at[pl.ds(tc * 8, 8)],
                    out_hbm.at[h, pl.ds(tc * 8, 8), pl.ds(bcol, CT)],
                    ssems[s],
                )
                for tc in range(D_MODEL // 8)
            ]

        def pre(h, s):
            @pl.when(h >= NB)
            def _():
                for d in scatter_descs(h - NB, s):
                    d.wait()

            for g in range(tg):
                sl = pl.ds(g * LANES, LANES)
                v = idx_all[h, sl]
                pvs[s][sl] = v >> 1
                hvs[s][sl] = (v & 1) << 6
            gather_desc(s).start()

        def post(h, s):
            gather_desc(s).wait()
            buf, tbuf, hv_ref = bufs[s], tbufs[s], hvs[s]
            for g in range(tg):
                rowv = lax.iota(jnp.int32, LANES) + (g * LANES)
                hv = hv_ref[pl.ds(g * LANES, LANES)]

                @plsc.parallel_loop(0, D_MODEL, unroll=8)
                def _sel(c):
                    cv = jnp.full((LANES,), c, jnp.int32)
                    v = plsc.load_gather(buf, [rowv, hv + cv])
                    tbuf[c, pl.ds(g * LANES, LANES)] = v * SCALE

            for d in scatter_descs(h, s):
                d.start()

        def step(h, s_pre, s_post):
            @pl.when(h < nch)
            def _():
                pre(h, s_pre)

            h2 = h - 2

            @pl.when(jnp.logical_and(h2 >= 0, h2 < nch))
            def _():
                post(h2, s_post)

        n_steps = nch + 2
        n_rounds = (n_steps + NB - 1) // NB

        def round_body(k, carry):
            for t in range(NB):
                step(k * NB + t, t, (t + 2) % NB)
            return carry

        lax.fori_loop(0, n_rounds, round_body, 0)

        for h in range(nch - NB, nch):
            for d in scatter_descs(h, h % NB):
                d.wait()

    return body


def kernel(token_ids, embedding_weights):
    BATCH, HIST = token_ids.shape
    V = embedding_weights.shape[0]
    tab2 = embedding_weights.reshape(V // 2, 2 * D_MODEL)
    idxT = token_ids.T
    out3 = _make_sc_gather(BATCH, HIST, V)(tab2, idxT)
    return out3.transpose(2, 0, 1)


# speedup vs baseline: 1.4191x; 1.0132x over previous
import functools
import math

import jax
import jax.numpy as jnp
from jax import lax
from jax.experimental import pallas as pl
from jax.experimental.pallas import tpu as pltpu
from jax.experimental.pallas import tpu_sc as plsc

D_MODEL = 64
SCALE = 8.0
LANES = 16
CT = 128
NB = 4


def _make_sc_gather(BATCH, HIST, V):
    info = plsc.get_sparse_core_info()
    NC, NS = info.num_cores, info.num_subcores
    NW = NC * NS
    assert BATCH % (CT * NW) == 0
    nch = HIST
    tg = CT // LANES

    mesh = plsc.VectorSubcoreMesh(core_axis_name="c", subcore_axis_name="s")

    @functools.partial(
        pl.kernel,
        out_type=jax.ShapeDtypeStruct((HIST, D_MODEL, BATCH), jnp.float32),
        mesh=mesh,
        scratch_types=[
            pltpu.VMEM((nch, CT), jnp.int32),
            [pltpu.VMEM((CT,), jnp.int32) for _ in range(NB)],
            [pltpu.VMEM((CT,), jnp.int32) for _ in range(NB)],
            [pltpu.VMEM((CT, 2 * D_MODEL), jnp.float32) for _ in range(NB)],
            [pltpu.VMEM((D_MODEL, CT), jnp.float32) for _ in range(NB)],
            [pltpu.SemaphoreType.DMA for _ in range(NB)],
            [pltpu.SemaphoreType.DMA for _ in range(NB)],
        ],
        compiler_params=pltpu.CompilerParams(needs_layout_passes=False),
    )
    def body(tab_hbm, idx_hbm, out_hbm, idx_all, pvs, hvs, bufs, tbufs,
             gsems, ssems):
        wid = lax.axis_index("s") * NC + lax.axis_index("c")
        bcol = wid * CT

        for th in range(nch // 8):
            pltpu.sync_copy(
                idx_hbm.at[pl.ds(th * 8, 8), pl.ds(bcol, CT)],
                idx_all.at[pl.ds(th * 8, 8)],
            )

        def gather_desc(s):
            return pltpu.make_async_copy(tab_hbm.at[pvs[s]], bufs[s], gsems[s])

        def scatter_descs(h, s):
            return [
                pltpu.make_async_copy(
                    tbufs[s].at[pl.ds(tc * 8, 8)],
                    out_hbm.at[h, pl.ds(tc * 8, 8), pl.ds(bcol, CT)],
                    ssems[s],
                )
                for tc in range(D_MODEL // 8)
            ]

        def pre(h, s):
            @pl.when(h >= NB)
            def _():
                for d in scatter_descs(h - NB, s):
                    d.wait()

            for g in range(tg):
                sl = pl.ds(g * LANES, LANES)
                v = idx_all[h, sl]
                pvs[s][sl] = v >> 1
                hvs[s][sl] = (v & 1) << 6
            gather_desc(s).start()

        def post(h, s):
            gather_desc(s).wait()
            buf, tbuf, hv_ref = bufs[s], tbufs[s], hvs[s]
            rowvs = [lax.iota(jnp.int32, LANES) + (g * LANES) for g in range(tg)]
            hvv = [hv_ref[pl.ds(g * LANES, LANES)] for g in range(tg)]

            @plsc.parallel_loop(0, D_MODEL, unroll=2)
            def _sel(c):
                cv = jnp.full((LANES,), c, jnp.int32)
                for g in range(tg):
                    v = plsc.load_gather(buf, [rowvs[g], hvv[g] + cv])
                    tbuf[c, pl.ds(g * LANES, LANES)] = v * SCALE

            for d in scatter_descs(h, s):
                d.start()

        def step(h, s_pre, s_post):
            @pl.when(h < nch)
            def _():
                pre(h, s_pre)

            h2 = h - 2

            @pl.when(jnp.logical_and(h2 >= 0, h2 < nch))
            def _():
                post(h2, s_post)

        n_steps = nch + 2
        n_rounds = (n_steps + NB - 1) // NB

        def round_body(k, carry):
            for t in range(NB):
                step(k * NB + t, t, (t + 2) % NB)
            return carry

        lax.fori_loop(0, n_rounds, round_body, 0)

        for h in range(nch - NB, nch):
            for d in scatter_descs(h, h % NB):
                d.wait()

    return body


def kernel(token_ids, embedding_weights):
    BATCH, HIST = token_ids.shape
    V = embedding_weights.shape[0]
    tab2 = embedding_weights.reshape(V // 2, 2 * D_MODEL)
    idxT = token_ids.T
    out3 = _make_sc_gather(BATCH, HIST, V)(tab2, idxT)
    return out3.transpose(2, 0, 1)
